# Initial kernel scaffold; baseline (speedup 1.0000x reference)
import jax
import jax.numpy as jnp
from jax.experimental import pallas as pl


def kernel(positions, cell, numbers, noise_eps):
    raise NotImplementedError("write your pallas kernel here")


# speedup vs baseline: 10.5447x; 10.5447x over previous
from math import ceil

import jax
import jax.numpy as jnp
from jax.experimental import pallas as pl
from jax.experimental.pallas import tpu as pltpu

_K = 17
_N_TARGET = 2000
_NOISE = 0.5

_RB = 128


def _knn_body(xr_ref, yr_ref, zr_ref, xs_ref, ys_ref, zs_ref, vals_ref, idx_ref):
    i = pl.program_id(0)
    S = xs_ref.shape[1]
    rx = xr_ref[0, :].reshape(_RB, 1)
    ry = yr_ref[0, :].reshape(_RB, 1)
    rz = zr_ref[0, :].reshape(_RB, 1)
    dx = rx - xs_ref[0, :].reshape(1, S)
    dy = ry - ys_ref[0, :].reshape(1, S)
    dz = rz - zs_ref[0, :].reshape(1, S)
    d2 = (dx * dx + dy * dy) + dz * dz
    cols = jax.lax.broadcasted_iota(jnp.int32, (_RB, S), 1)
    rows_g = jax.lax.broadcasted_iota(jnp.int32, (_RB, S), 0) + i * _RB
    d2 = jnp.where(cols == rows_g, jnp.float32(1e9), d2)

    vals = []
    idxs = []
    big = jnp.float32(jnp.inf)
    for _ in range(_K):
        m = jnp.min(d2, axis=1, keepdims=True)
        ii = jnp.min(jnp.where(d2 == m, cols, S), axis=1, keepdims=True)
        vals.append(m)
        idxs.append(ii)
        d2 = jnp.where(cols == ii, big, d2)
    v = jnp.concatenate(vals, axis=1)
    vals_ref[...] = jnp.sqrt(jnp.maximum(v, jnp.float32(1e-12)))
    idx_ref[...] = jnp.concatenate(idxs, axis=1)


def kernel(positions, cell, numbers, noise_eps):
    frac = positions @ jnp.linalg.inv(cell)
    n = positions.shape[0]
    replicates = ceil((_N_TARGET / n) ** (1.0 / 3.0))
    r = replicates
    ax = jnp.arange(r, dtype=frac.dtype)
    offs = jnp.stack(jnp.meshgrid(ax, ax, ax, indexing="ij"), axis=-1).reshape(-1, 3)
    sc = (frac[None, :, :] + offs[:, None, :]).reshape(-1, 3)
    sc = sc + _NOISE * noise_eps
    cart = sc @ cell
    S = cart.shape[0]

    xs = cart[:, 0].reshape(1, S)
    ys = cart[:, 1].reshape(1, S)
    zs = cart[:, 2].reshape(1, S)

    nblk = S // _RB
    row_spec = pl.BlockSpec((1, _RB), lambda i: (0, i))
    full_spec = pl.BlockSpec((1, S), lambda i: (0, 0))
    out_spec = pl.BlockSpec((_RB, _K), lambda i: (i, 0))

    dists, idx = pl.pallas_call(
        _knn_body,
        grid=(nblk,),
        in_specs=[row_spec, row_spec, row_spec, full_spec, full_spec, full_spec],
        out_specs=[out_spec, out_spec],
        out_shape=[
            jax.ShapeDtypeStruct((S, _K), jnp.float32),
            jax.ShapeDtypeStruct((S, _K), jnp.int32),
        ],
    )(xs, ys, zs, xs, ys, zs)

    src = jnp.repeat(jnp.arange(S, dtype=jnp.int32), _K)
    dst = idx.reshape(-1)
    numbers_rep = jnp.tile(numbers, r ** 3)
    return dists, src, dst, numbers_rep, jnp.float32(_NOISE)
